# baseline (speedup 1.0000x reference)
import functools

import jax
import jax.numpy as jnp
from jax import lax
from jax.experimental import pallas as pl

N_ATOMS = 10000
N_EDGES = 320000
N_MOL = 10000
NB = 128
NG = 25
N_CONV = 3
CUTOFF = 5.0

NPAD = 10240
EPAD = 321536
ROW_T = 128
EDGE_T = 512


def _ssp(x):
    return jnp.maximum(x, 0.0) + jnp.log1p(jnp.exp(-jnp.abs(x))) - 0.6931471805599453


def _embed_body(z_ref, emb_ref, out_ref):
    z = z_ref[...].astype(jnp.int32)
    ids = lax.broadcasted_iota(jnp.int32, (ROW_T, 100), 1)
    onehot = jnp.where(ids == z, 1.0, 0.0)
    out_ref[...] = jnp.dot(onehot, emb_ref[...], preferred_element_type=jnp.float32)


def _embed(zf, emb):
    return pl.pallas_call(
        _embed_body,
        grid=(NPAD // ROW_T,),
        in_specs=[
            pl.BlockSpec((ROW_T, 1), lambda i: (i, 0)),
            pl.BlockSpec((100, NB), lambda i: (0, 0)),
        ],
        out_specs=pl.BlockSpec((ROW_T, NB), lambda i: (i, 0)),
        out_shape=jax.ShapeDtypeStruct((NPAD, NB), jnp.float32),
    )(zf, emb)


def _filter_body(d2_ref, we1_ref, be1_ref, we2_ref, be2_ref, out_ref):
    d = jnp.sqrt(d2_ref[...])
    offsets = (lax.broadcasted_iota(jnp.int32, (1, NG), 1).astype(jnp.float32)
               * (CUTOFF / (NG - 1)))
    width = CUTOFF / (NG - 1)
    coeff = -0.5 / (width * width)
    diff = d - offsets
    e_sm = jnp.exp(coeff * diff * diff)
    ef = _ssp(jnp.dot(e_sm, we1_ref[...], preferred_element_type=jnp.float32)
              + be1_ref[...])
    w = jnp.dot(ef, we2_ref[...], preferred_element_type=jnp.float32) + be2_ref[...]
    out_ref[0] = w[:, :64]
    out_ref[1] = w[:, 64:]


def _edge_filter(d2, we1, be1, we2, be2):
    return pl.pallas_call(
        _filter_body,
        grid=(EPAD // EDGE_T,),
        in_specs=[
            pl.BlockSpec((EDGE_T, 1), lambda i: (i, 0)),
            pl.BlockSpec((NG, NG), lambda i: (0, 0)),
            pl.BlockSpec((1, NG), lambda i: (0, 0)),
            pl.BlockSpec((NG, NB), lambda i: (0, 0)),
            pl.BlockSpec((1, NB), lambda i: (0, 0)),
        ],
        out_specs=pl.BlockSpec((2, EDGE_T, 64), lambda i: (0, i, 0)),
        out_shape=jax.ShapeDtypeStruct((2, EPAD, 64), jnp.float32),
    )(d2, we1, be1, we2, be2)


def _node_linear_body(r_ref, w_ref, b_ref, out_ref):
    rn = jnp.dot(r_ref[...], w_ref[...], preferred_element_type=jnp.float32) + b_ref[...]
    out_ref[0] = rn[:, :64]
    out_ref[1] = rn[:, 64:]


def _node_linear(r, w, b):
    return pl.pallas_call(
        _node_linear_body,
        grid=(NPAD // ROW_T,),
        in_specs=[
            pl.BlockSpec((ROW_T, NB), lambda i: (i, 0)),
            pl.BlockSpec((NB, NB), lambda i: (0, 0)),
            pl.BlockSpec((1, NB), lambda i: (0, 0)),
        ],
        out_specs=pl.BlockSpec((2, ROW_T, 64), lambda i: (0, i, 0)),
        out_shape=jax.ShapeDtypeStruct((2, NPAD, 64), jnp.float32),
    )(r, w, b)


def _update_body(agg_ref, r_ref, w1_ref, b1_ref, w2_ref, b2_ref, out_ref):
    a = jnp.concatenate([agg_ref[0], agg_ref[1]], axis=1)
    h = _ssp(jnp.dot(a, w1_ref[...], preferred_element_type=jnp.float32) + b1_ref[...])
    out_ref[...] = (r_ref[...]
                    + jnp.dot(h, w2_ref[...], preferred_element_type=jnp.float32)
                    + b2_ref[...])


def _update(agg, r, w1, b1, w2, b2):
    return pl.pallas_call(
        _update_body,
        grid=(NPAD // ROW_T,),
        in_specs=[
            pl.BlockSpec((2, ROW_T, 64), lambda i: (0, i, 0)),
            pl.BlockSpec((ROW_T, NB), lambda i: (i, 0)),
            pl.BlockSpec((NB, NB), lambda i: (0, 0)),
            pl.BlockSpec((1, NB), lambda i: (0, 0)),
            pl.BlockSpec((NB, NB), lambda i: (0, 0)),
            pl.BlockSpec((1, NB), lambda i: (0, 0)),
        ],
        out_specs=pl.BlockSpec((ROW_T, NB), lambda i: (i, 0)),
        out_shape=jax.ShapeDtypeStruct((NPAD, NB), jnp.float32),
    )(agg, r, w1, b1, w2, b2)


def _head_body(r_ref, w1_ref, b1_ref, w2_ref, b2_ref, out_ref):
    h = _ssp(jnp.dot(r_ref[...], w1_ref[...], preferred_element_type=jnp.float32)
             + b1_ref[...])
    out_ref[...] = jnp.dot(h, w2_ref[...], preferred_element_type=jnp.float32) + b2_ref[...]


def _head(r, w1, b1, w2, b2):
    return pl.pallas_call(
        _head_body,
        grid=(NPAD // ROW_T,),
        in_specs=[
            pl.BlockSpec((ROW_T, NB), lambda i: (i, 0)),
            pl.BlockSpec((NB, 64), lambda i: (0, 0)),
            pl.BlockSpec((1, 64), lambda i: (0, 0)),
            pl.BlockSpec((64, 1), lambda i: (0, 0)),
            pl.BlockSpec((1, 1), lambda i: (0, 0)),
        ],
        out_specs=pl.BlockSpec((ROW_T, 1), lambda i: (i, 0)),
        out_shape=jax.ShapeDtypeStruct((NPAD, 1), jnp.float32),
    )(r, w1, b1, w2, b2)


def kernel(nxyz, num_atoms, nbr_list, emb, W_e1, b_e1, W_e2, b_e2, W_n, b_n,
           W_u1, b_u1, W_u2, b_u2, W_r1, b_r1, W_r2, b_r2):
    z = nxyz[:, 0]
    xyz = nxyz[:, 1:4]
    a0 = nbr_list[:, 0].astype(jnp.int32)
    a1 = nbr_list[:, 1].astype(jnp.int32)

    pad_e = EPAD - N_EDGES
    a0p = jnp.concatenate([a0, jnp.full((pad_e,), N_ATOMS, jnp.int32)])
    a1p = jnp.concatenate([a1, jnp.full((pad_e,), N_ATOMS, jnp.int32)])

    zf = jnp.pad(z, (0, NPAD - N_ATOMS)).reshape(NPAD, 1).astype(jnp.float32)
    xyzp = jnp.pad(xyz, ((0, NPAD - N_ATOMS), (0, 0)))

    dvec = xyzp[a0p] - xyzp[a1p]
    d2 = jnp.sum(dvec * dvec, axis=1, keepdims=True)

    r = _embed(zf, emb)

    for i in range(N_CONV):
        wij = _edge_filter(d2, W_e1[i], b_e1[i][None], W_e2[i], b_e2[i][None])
        rn = _node_linear(r, W_n[i], b_n[i][None])

        rn_full = jnp.concatenate([rn[0], rn[1]], axis=1)
        wij_full = jnp.concatenate([wij[0], wij[1]], axis=1)
        m_ij = rn_full[a0p] * wij_full
        m_ji = rn_full[a1p] * wij_full
        agg_full = jax.ops.segment_sum(m_ij, a1p, num_segments=NPAD)
        agg_full = agg_full + jax.ops.segment_sum(m_ji, a0p, num_segments=NPAD)
        agg = jnp.stack([agg_full[:, :64], agg_full[:, 64:]])

        r = _update(agg, r, W_u1[i], b_u1[i][None], W_u2[i], b_u2[i][None])

    atom_e = _head(r, W_r1, b_r1[None], W_r2, b_r2[None])
    return atom_e[:N_MOL, 0]


# speedup vs baseline: 1.0333x; 1.0333x over previous
import functools

import jax
import jax.numpy as jnp
from jax import lax
from jax.experimental import pallas as pl
from jax.experimental.pallas import tpu as pltpu
from jax.experimental.pallas import tpu_sc as plsc

N_ATOMS = 10000
N_EDGES = 320000
N_MOL = 10000
NB = 128
NG = 25
N_CONV = 3
CUTOFF = 5.0

NPAD = 10240
EPAD = 327680
ROW_T = 128
EDGE_T = 512

CH = 128
NCH = EPAD // (16 * CH)
EPT = NCH * CH
WIN = 8
NWIN = NCH // WIN
RPT = NPAD // 16


def _ssp(x):
    return jnp.maximum(x, 0.0) + jnp.log1p(jnp.exp(-jnp.abs(x))) - 0.6931471805599453


def _embed_body(z_ref, emb_ref, out_ref):
    z = z_ref[...].astype(jnp.int32)
    ids = lax.broadcasted_iota(jnp.int32, (ROW_T, 100), 1)
    onehot = jnp.where(ids == z, 1.0, 0.0)
    out_ref[...] = jnp.dot(onehot, emb_ref[...], preferred_element_type=jnp.float32)


def _embed(zf, emb):
    return pl.pallas_call(
        _embed_body,
        grid=(NPAD // ROW_T,),
        in_specs=[
            pl.BlockSpec((ROW_T, 1), lambda i: (i, 0)),
            pl.BlockSpec((100, NB), lambda i: (0, 0)),
        ],
        out_specs=pl.BlockSpec((ROW_T, NB), lambda i: (i, 0)),
        out_shape=jax.ShapeDtypeStruct((NPAD, NB), jnp.float32),
    )(zf, emb)


def _filter_body(d2_ref, we1_ref, be1_ref, we2_ref, be2_ref, out_ref):
    d = jnp.sqrt(d2_ref[...])
    offsets = (lax.broadcasted_iota(jnp.int32, (1, NG), 1).astype(jnp.float32)
               * (CUTOFF / (NG - 1)))
    width = CUTOFF / (NG - 1)
    coeff = -0.5 / (width * width)
    diff = d - offsets
    e_sm = jnp.exp(coeff * diff * diff)
    ef = _ssp(jnp.dot(e_sm, we1_ref[...], preferred_element_type=jnp.float32)
              + be1_ref[...])
    w = jnp.dot(ef, we2_ref[...], preferred_element_type=jnp.float32) + be2_ref[...]
    out_ref[0] = w[:, :64]
    out_ref[1] = w[:, 64:]


def _edge_filter(d2, we1, be1, we2, be2):
    return pl.pallas_call(
        _filter_body,
        grid=(EPAD // EDGE_T,),
        in_specs=[
            pl.BlockSpec((EDGE_T, 1), lambda i: (i, 0)),
            pl.BlockSpec((NG, NG), lambda i: (0, 0)),
            pl.BlockSpec((1, NG), lambda i: (0, 0)),
            pl.BlockSpec((NG, NB), lambda i: (0, 0)),
            pl.BlockSpec((1, NB), lambda i: (0, 0)),
        ],
        out_specs=pl.BlockSpec((2, EDGE_T, 64), lambda i: (0, i, 0)),
        out_shape=jax.ShapeDtypeStruct((2, EPAD, 64), jnp.float32),
    )(d2, we1, be1, we2, be2)


def _node_linear_body(r_ref, w_ref, b_ref, out_ref):
    out_ref[...] = (jnp.dot(r_ref[...], w_ref[...], preferred_element_type=jnp.float32)
                    + b_ref[...])


def _node_linear(r, w, b):
    return pl.pallas_call(
        _node_linear_body,
        grid=(NPAD // ROW_T,),
        in_specs=[
            pl.BlockSpec((ROW_T, NB), lambda i: (i, 0)),
            pl.BlockSpec((NB, NB), lambda i: (0, 0)),
            pl.BlockSpec((1, NB), lambda i: (0, 0)),
        ],
        out_specs=pl.BlockSpec((ROW_T, NB), lambda i: (i, 0)),
        out_shape=jax.ShapeDtypeStruct((NPAD, NB), jnp.float32),
    )(r, w, b)


def _update_body(agg_ref, r_ref, w1_ref, b1_ref, w2_ref, b2_ref, out_ref):
    a = jnp.concatenate([agg_ref[0], agg_ref[1]], axis=1)
    h = _ssp(jnp.dot(a, w1_ref[...], preferred_element_type=jnp.float32) + b1_ref[...])
    out_ref[...] = (r_ref[...]
                    + jnp.dot(h, w2_ref[...], preferred_element_type=jnp.float32)
                    + b2_ref[...])


def _update(agg, r, w1, b1, w2, b2):
    return pl.pallas_call(
        _update_body,
        grid=(NPAD // ROW_T,),
        in_specs=[
            pl.BlockSpec((2, ROW_T, 64), lambda i: (0, i, 0)),
            pl.BlockSpec((ROW_T, NB), lambda i: (i, 0)),
            pl.BlockSpec((NB, NB), lambda i: (0, 0)),
            pl.BlockSpec((1, NB), lambda i: (0, 0)),
            pl.BlockSpec((NB, NB), lambda i: (0, 0)),
            pl.BlockSpec((1, NB), lambda i: (0, 0)),
        ],
        out_specs=pl.BlockSpec((ROW_T, NB), lambda i: (i, 0)),
        out_shape=jax.ShapeDtypeStruct((NPAD, NB), jnp.float32),
    )(agg, r, w1, b1, w2, b2)


def _head_body(r_ref, w1_ref, b1_ref, w2_ref, b2_ref, out_ref):
    h = _ssp(jnp.dot(r_ref[...], w1_ref[...], preferred_element_type=jnp.float32)
             + b1_ref[...])
    out_ref[...] = jnp.dot(h, w2_ref[...], preferred_element_type=jnp.float32) + b2_ref[...]


def _head(r, w1, b1, w2, b2):
    return pl.pallas_call(
        _head_body,
        grid=(NPAD // ROW_T,),
        in_specs=[
            pl.BlockSpec((ROW_T, NB), lambda i: (i, 0)),
            pl.BlockSpec((NB, 64), lambda i: (0, 0)),
            pl.BlockSpec((1, 64), lambda i: (0, 0)),
            pl.BlockSpec((64, 1), lambda i: (0, 0)),
            pl.BlockSpec((1, 1), lambda i: (0, 0)),
        ],
        out_specs=pl.BlockSpec((ROW_T, 1), lambda i: (i, 0)),
        out_shape=jax.ShapeDtypeStruct((NPAD, 1), jnp.float32),
    )(r, w1, b1, w2, b2)


NCHC = NCH // 2


def _d2_body(x_hbm, y_hbm, z_hbm, i0_hbm, i1_hbm, out_hbm,
             i0_v, i1_v, xa, ya, za, xb, yb, zb, d2_v):
    c = lax.axis_index("c")
    s = lax.axis_index("s")

    pltpu.sync_copy(i0_hbm.at[s, pl.ds(c * NCHC, NCHC)], i0_v)
    pltpu.sync_copy(i1_hbm.at[s, pl.ds(c * NCHC, NCHC)], i1_v)

    def chunkfn(j, carry):
        pltpu.sync_copy(x_hbm.at[i0_v.at[j]], xa)
        pltpu.sync_copy(y_hbm.at[i0_v.at[j]], ya)
        pltpu.sync_copy(z_hbm.at[i0_v.at[j]], za)
        pltpu.sync_copy(x_hbm.at[i1_v.at[j]], xb)
        pltpu.sync_copy(y_hbm.at[i1_v.at[j]], yb)
        pltpu.sync_copy(z_hbm.at[i1_v.at[j]], zb)
        for k in range(8):
            sl = pl.ds(k * 16, 16)
            dx = xa[sl] - xb[sl]
            dy = ya[sl] - yb[sl]
            dz = za[sl] - zb[sl]
            d2_v[j, sl] = dx * dx + dy * dy + dz * dz
        return carry

    lax.fori_loop(0, NCHC, chunkfn, 0)
    pltpu.sync_copy(d2_v, out_hbm.at[s, pl.ds(c * NCHC, NCHC)])


@functools.cache
def _d2_kernel():
    mesh = plsc.VectorSubcoreMesh(core_axis_name="c", subcore_axis_name="s")
    return pl.kernel(
        _d2_body,
        out_type=jax.ShapeDtypeStruct((16, NCH, CH), jnp.float32),
        mesh=mesh,
        scratch_types=[
            pltpu.VMEM((NCHC, CH), jnp.int32),
            pltpu.VMEM((NCHC, CH), jnp.int32),
            pltpu.VMEM((CH,), jnp.float32),
            pltpu.VMEM((CH,), jnp.float32),
            pltpu.VMEM((CH,), jnp.float32),
            pltpu.VMEM((CH,), jnp.float32),
            pltpu.VMEM((CH,), jnp.float32),
            pltpu.VMEM((CH,), jnp.float32),
            pltpu.VMEM((NCHC, CH), jnp.float32),
        ],
    )


def _msg_body(rn_hbm, wij_hbm, idx0_hbm, idx1_hbm, out_hbm,
              idx0_v, idx1_v, av, bv, wv, pv, qv):
    c = lax.axis_index("c")
    s = lax.axis_index("s")

    def window(wd, carry):
        pltpu.sync_copy(idx0_hbm.at[s, pl.ds(wd * WIN, WIN)], idx0_v)
        pltpu.sync_copy(idx1_hbm.at[s, pl.ds(wd * WIN, WIN)], idx1_v)

        def chunk(jj, inner):
            base = s * EPT + (wd * WIN + jj) * CH
            pltpu.sync_copy(wij_hbm.at[c, pl.ds(base, CH)], wv)
            pltpu.sync_copy(rn_hbm.at[idx0_v.at[jj]], av)
            pltpu.sync_copy(rn_hbm.at[idx1_v.at[jj]], bv)

            def row(rr, acc):
                for k in range(4):
                    sl = pl.ds(k * 16, 16)
                    slc = pl.ds(c * 64 + k * 16, 16)
                    wk = wv[rr, sl]
                    pv[rr, sl] = av[rr, slc] * wk
                    qv[rr, sl] = bv[rr, slc] * wk
                return acc
            lax.fori_loop(0, CH, row, 0)

            pltpu.sync_copy(pv, out_hbm.at[c, 0, pl.ds(base, CH)])
            pltpu.sync_copy(qv, out_hbm.at[c, 1, pl.ds(base, CH)])
            return inner
        lax.fori_loop(0, WIN, chunk, 0)
        return carry

    lax.fori_loop(0, NWIN, window, 0)


@functools.cache
def _msg_kernel():
    mesh = plsc.VectorSubcoreMesh(core_axis_name="c", subcore_axis_name="s")
    return pl.kernel(
        _msg_body,
        out_type=jax.ShapeDtypeStruct((2, 2, EPAD, 64), jnp.float32),
        mesh=mesh,
        scratch_types=[
            pltpu.VMEM((WIN, CH), jnp.int32),
            pltpu.VMEM((WIN, CH), jnp.int32),
            pltpu.VMEM((CH, NB), jnp.float32),
            pltpu.VMEM((CH, NB), jnp.float32),
            pltpu.VMEM((CH, 64), jnp.float32),
            pltpu.VMEM((CH, 64), jnp.float32),
            pltpu.VMEM((CH, 64), jnp.float32),
        ],
    )


def kernel(nxyz, num_atoms, nbr_list, emb, W_e1, b_e1, W_e2, b_e2, W_n, b_n,
           W_u1, b_u1, W_u2, b_u2, W_r1, b_r1, W_r2, b_r2):
    z = nxyz[:, 0]
    xyz = nxyz[:, 1:4]
    a0 = nbr_list[:, 0].astype(jnp.int32)
    a1 = nbr_list[:, 1].astype(jnp.int32)

    pad_e = EPAD - N_EDGES
    a0p = jnp.concatenate([a0, jnp.full((pad_e,), N_ATOMS, jnp.int32)])
    a1p = jnp.concatenate([a1, jnp.full((pad_e,), N_ATOMS, jnp.int32)])

    zf = jnp.pad(z, (0, NPAD - N_ATOMS)).reshape(NPAD, 1).astype(jnp.float32)
    xyzp = jnp.pad(xyz, ((0, NPAD - N_ATOMS), (0, 0)))

    i0_t = a0p.reshape(16, NCH, CH)
    i1_t = a1p.reshape(16, NCH, CH)
    d2 = _d2_kernel()(xyzp[:, 0], xyzp[:, 1], xyzp[:, 2],
                      i0_t, i1_t).reshape(EPAD, 1)

    r = _embed(zf, emb)

    for i in range(N_CONV):
        wij = _edge_filter(d2, W_e1[i], b_e1[i][None], W_e2[i], b_e2[i][None])
        rn = _node_linear(r, W_n[i], b_n[i][None])

        pq = _msg_kernel()(rn, wij, i0_t, i1_t)
        m_ij = jnp.concatenate([pq[0, 0], pq[1, 0]], axis=1)
        m_ji = jnp.concatenate([pq[0, 1], pq[1, 1]], axis=1)
        agg_full = jax.ops.segment_sum(m_ij, a1p, num_segments=NPAD)
        agg_full = agg_full + jax.ops.segment_sum(m_ji, a0p, num_segments=NPAD)
        agg = jnp.stack([agg_full[:, :64], agg_full[:, 64:]])

        r = _update(agg, r, W_u1[i], b_u1[i][None], W_u2[i], b_u2[i][None])

    atom_e = _head(r, W_r1, b_r1[None], W_r2, b_r2[None])
    return atom_e[:N_MOL, 0]


# speedup vs baseline: 1.1336x; 1.0971x over previous
import functools

import jax
import jax.numpy as jnp
from jax import lax
from jax.experimental import pallas as pl
from jax.experimental.pallas import tpu as pltpu
from jax.experimental.pallas import tpu_sc as plsc

N_ATOMS = 10000
N_EDGES = 320000
N_MOL = 10000
NB = 128
NG = 25
N_CONV = 3
CUTOFF = 5.0

NPAD = 10240
EPAD = 327680
ROW_T = 128
EDGE_T = 512

CH = 128
NCH = EPAD // (16 * CH)
EPT = NCH * CH
WIN = 8
NWIN = NCH // WIN
RPT = NPAD // 16


def _ssp(x):
    return jnp.maximum(x, 0.0) + jnp.log1p(jnp.exp(-jnp.abs(x))) - 0.6931471805599453


def _embed_body(z_ref, emb_ref, out_ref):
    z = z_ref[...].astype(jnp.int32)
    ids = lax.broadcasted_iota(jnp.int32, (ROW_T, 100), 1)
    onehot = jnp.where(ids == z, 1.0, 0.0)
    out_ref[...] = jnp.dot(onehot, emb_ref[...], preferred_element_type=jnp.float32)


def _embed(zf, emb):
    return pl.pallas_call(
        _embed_body,
        grid=(NPAD // ROW_T,),
        in_specs=[
            pl.BlockSpec((ROW_T, 1), lambda i: (i, 0)),
            pl.BlockSpec((100, NB), lambda i: (0, 0)),
        ],
        out_specs=pl.BlockSpec((ROW_T, NB), lambda i: (i, 0)),
        out_shape=jax.ShapeDtypeStruct((NPAD, NB), jnp.float32),
    )(zf, emb)


def _filter_body(d2_ref, we1_ref, be1_ref, we2_ref, be2_ref, out_ref):
    d = jnp.sqrt(d2_ref[...])
    offsets = (lax.broadcasted_iota(jnp.int32, (1, NG), 1).astype(jnp.float32)
               * (CUTOFF / (NG - 1)))
    width = CUTOFF / (NG - 1)
    coeff = -0.5 / (width * width)
    diff = d - offsets
    e_sm = jnp.exp(coeff * diff * diff)
    ef = _ssp(jnp.dot(e_sm, we1_ref[...], preferred_element_type=jnp.float32)
              + be1_ref[...])
    w = jnp.dot(ef, we2_ref[...], preferred_element_type=jnp.float32) + be2_ref[...]
    out_ref[0] = w[:, :64]
    out_ref[1] = w[:, 64:]


def _edge_filter(d2, we1, be1, we2, be2):
    return pl.pallas_call(
        _filter_body,
        grid=(EPAD // EDGE_T,),
        in_specs=[
            pl.BlockSpec((EDGE_T, 1), lambda i: (i, 0)),
            pl.BlockSpec((NG, NG), lambda i: (0, 0)),
            pl.BlockSpec((1, NG), lambda i: (0, 0)),
            pl.BlockSpec((NG, NB), lambda i: (0, 0)),
            pl.BlockSpec((1, NB), lambda i: (0, 0)),
        ],
        out_specs=pl.BlockSpec((2, EDGE_T, 64), lambda i: (0, i, 0)),
        out_shape=jax.ShapeDtypeStruct((2, EPAD, 64), jnp.float32),
    )(d2, we1, be1, we2, be2)


def _node_linear_body(r_ref, w_ref, b_ref, out_ref):
    out_ref[...] = (jnp.dot(r_ref[...], w_ref[...], preferred_element_type=jnp.float32)
                    + b_ref[...])


def _node_linear(r, w, b):
    return pl.pallas_call(
        _node_linear_body,
        grid=(NPAD // ROW_T,),
        in_specs=[
            pl.BlockSpec((ROW_T, NB), lambda i: (i, 0)),
            pl.BlockSpec((NB, NB), lambda i: (0, 0)),
            pl.BlockSpec((1, NB), lambda i: (0, 0)),
        ],
        out_specs=pl.BlockSpec((ROW_T, NB), lambda i: (i, 0)),
        out_shape=jax.ShapeDtypeStruct((NPAD, NB), jnp.float32),
    )(r, w, b)


def _update_body(agg_ref, r_ref, w1_ref, b1_ref, w2_ref, b2_ref, out_ref):
    a = jnp.concatenate([agg_ref[0], agg_ref[1]], axis=1)
    h = _ssp(jnp.dot(a, w1_ref[...], preferred_element_type=jnp.float32) + b1_ref[...])
    out_ref[...] = (r_ref[...]
                    + jnp.dot(h, w2_ref[...], preferred_element_type=jnp.float32)
                    + b2_ref[...])


def _update(agg, r, w1, b1, w2, b2):
    return pl.pallas_call(
        _update_body,
        grid=(NPAD // ROW_T,),
        in_specs=[
            pl.BlockSpec((2, ROW_T, 64), lambda i: (0, i, 0)),
            pl.BlockSpec((ROW_T, NB), lambda i: (i, 0)),
            pl.BlockSpec((NB, NB), lambda i: (0, 0)),
            pl.BlockSpec((1, NB), lambda i: (0, 0)),
            pl.BlockSpec((NB, NB), lambda i: (0, 0)),
            pl.BlockSpec((1, NB), lambda i: (0, 0)),
        ],
        out_specs=pl.BlockSpec((ROW_T, NB), lambda i: (i, 0)),
        out_shape=jax.ShapeDtypeStruct((NPAD, NB), jnp.float32),
    )(agg, r, w1, b1, w2, b2)


def _head_body(r_ref, w1_ref, b1_ref, w2_ref, b2_ref, out_ref):
    h = _ssp(jnp.dot(r_ref[...], w1_ref[...], preferred_element_type=jnp.float32)
             + b1_ref[...])
    out_ref[...] = jnp.dot(h, w2_ref[...], preferred_element_type=jnp.float32) + b2_ref[...]


def _head(r, w1, b1, w2, b2):
    return pl.pallas_call(
        _head_body,
        grid=(NPAD // ROW_T,),
        in_specs=[
            pl.BlockSpec((ROW_T, NB), lambda i: (i, 0)),
            pl.BlockSpec((NB, 64), lambda i: (0, 0)),
            pl.BlockSpec((1, 64), lambda i: (0, 0)),
            pl.BlockSpec((64, 1), lambda i: (0, 0)),
            pl.BlockSpec((1, 1), lambda i: (0, 0)),
        ],
        out_specs=pl.BlockSpec((ROW_T, 1), lambda i: (i, 0)),
        out_shape=jax.ShapeDtypeStruct((NPAD, 1), jnp.float32),
    )(r, w1, b1, w2, b2)


NCHC = NCH // 2


def _d2_body(x_hbm, y_hbm, z_hbm, i0_hbm, i1_hbm, out_hbm,
             i0_v, i1_v, xa, ya, za, xb, yb, zb, d2_v):
    c = lax.axis_index("c")
    s = lax.axis_index("s")

    pltpu.sync_copy(i0_hbm.at[s, pl.ds(c * NCHC, NCHC)], i0_v)
    pltpu.sync_copy(i1_hbm.at[s, pl.ds(c * NCHC, NCHC)], i1_v)

    def chunkfn(j, carry):
        pltpu.sync_copy(x_hbm.at[i0_v.at[j]], xa)
        pltpu.sync_copy(y_hbm.at[i0_v.at[j]], ya)
        pltpu.sync_copy(z_hbm.at[i0_v.at[j]], za)
        pltpu.sync_copy(x_hbm.at[i1_v.at[j]], xb)
        pltpu.sync_copy(y_hbm.at[i1_v.at[j]], yb)
        pltpu.sync_copy(z_hbm.at[i1_v.at[j]], zb)
        for k in range(8):
            sl = pl.ds(k * 16, 16)
            dx = xa[sl] - xb[sl]
            dy = ya[sl] - yb[sl]
            dz = za[sl] - zb[sl]
            d2_v[j, sl] = dx * dx + dy * dy + dz * dz
        return carry

    lax.fori_loop(0, NCHC, chunkfn, 0)
    pltpu.sync_copy(d2_v, out_hbm.at[s, pl.ds(c * NCHC, NCHC)])


@functools.cache
def _d2_kernel():
    mesh = plsc.VectorSubcoreMesh(core_axis_name="c", subcore_axis_name="s")
    return pl.kernel(
        _d2_body,
        out_type=jax.ShapeDtypeStruct((16, NCH, CH), jnp.float32),
        mesh=mesh,
        scratch_types=[
            pltpu.VMEM((NCHC, CH), jnp.int32),
            pltpu.VMEM((NCHC, CH), jnp.int32),
            pltpu.VMEM((CH,), jnp.float32),
            pltpu.VMEM((CH,), jnp.float32),
            pltpu.VMEM((CH,), jnp.float32),
            pltpu.VMEM((CH,), jnp.float32),
            pltpu.VMEM((CH,), jnp.float32),
            pltpu.VMEM((CH,), jnp.float32),
            pltpu.VMEM((NCHC, CH), jnp.float32),
        ],
    )


NCH2 = 2 * NCH
NWIN2 = NCH2 // WIN
EPT2 = 2 * EPT


def _msg_body(rn_hbm, wij_hbm, idx_hbm, out_hbm,
              idx_v, av, wv, pv, sga, sgw, swp):
    c = lax.axis_index("c")
    s = lax.axis_index("s")
    h = (s >= 8).astype(jnp.int32)

    def window(wd, carry):
        pltpu.sync_copy(idx_hbm.at[s, pl.ds(wd * WIN, WIN)], idx_v)

        def issue_gathers(jj):
            b = jj % 2
            base = s * EPT2 + (wd * WIN + jj) * CH
            base_w = base - h * EPAD
            return (
                pltpu.async_copy(wij_hbm.at[c, pl.ds(base_w, CH)], wv[b], sgw[b]),
                pltpu.async_copy(rn_hbm.at[idx_v.at[jj]], av[b], sga[b]),
            )

        gh = {0: issue_gathers(0)}
        wh = {}
        for jj in range(WIN):
            b = jj % 2
            base_w = s * EPT2 + (wd * WIN + jj) * CH - h * EPAD
            if jj + 1 < WIN:
                gh[jj + 1] = issue_gathers(jj + 1)
            for hd in gh.pop(jj):
                hd.wait()
            if jj - 2 in wh:
                wh.pop(jj - 2).wait()

            def row(rr, acc):
                for k in range(4):
                    sl = pl.ds(k * 16, 16)
                    slc = pl.ds(c * 64 + k * 16, 16)
                    pv[b][rr, sl] = av[b][rr, slc] * wv[b][rr, sl]
                return acc
            lax.fori_loop(0, CH, row, 0)

            wh[jj] = pltpu.async_copy(
                pv[b], out_hbm.at[c, h, pl.ds(base_w, CH)], swp[b])
        for hd in wh.values():
            hd.wait()
        return carry

    lax.fori_loop(0, NWIN2, window, 0)


@functools.cache
def _msg_kernel():
    mesh = plsc.VectorSubcoreMesh(core_axis_name="c", subcore_axis_name="s")
    return pl.kernel(
        _msg_body,
        out_type=jax.ShapeDtypeStruct((2, 2, EPAD, 64), jnp.float32),
        mesh=mesh,
        scratch_types=[
            pltpu.VMEM((WIN, CH), jnp.int32),
            [pltpu.VMEM((CH, NB), jnp.float32)] * 2,
            [pltpu.VMEM((CH, 64), jnp.float32)] * 2,
            [pltpu.VMEM((CH, 64), jnp.float32)] * 2,
            [pltpu.SemaphoreType.DMA] * 2,
            [pltpu.SemaphoreType.DMA] * 2,
            [pltpu.SemaphoreType.DMA] * 2,
        ],
    )


def kernel(nxyz, num_atoms, nbr_list, emb, W_e1, b_e1, W_e2, b_e2, W_n, b_n,
           W_u1, b_u1, W_u2, b_u2, W_r1, b_r1, W_r2, b_r2):
    z = nxyz[:, 0]
    xyz = nxyz[:, 1:4]
    a0 = nbr_list[:, 0].astype(jnp.int32)
    a1 = nbr_list[:, 1].astype(jnp.int32)

    pad_e = EPAD - N_EDGES
    a0p = jnp.concatenate([a0, jnp.full((pad_e,), N_ATOMS, jnp.int32)])
    a1p = jnp.concatenate([a1, jnp.full((pad_e,), N_ATOMS, jnp.int32)])

    zf = jnp.pad(z, (0, NPAD - N_ATOMS)).reshape(NPAD, 1).astype(jnp.float32)
    xyzp = jnp.pad(xyz, ((0, NPAD - N_ATOMS), (0, 0)))

    i0_t = a0p.reshape(16, NCH, CH)
    i1_t = a1p.reshape(16, NCH, CH)
    d2 = _d2_kernel()(xyzp[:, 0], xyzp[:, 1], xyzp[:, 2],
                      i0_t, i1_t).reshape(EPAD, 1)
    i_src = jnp.concatenate([a0p, a1p]).reshape(16, NCH2, CH)

    r = _embed(zf, emb)

    for i in range(N_CONV):
        wij = _edge_filter(d2, W_e1[i], b_e1[i][None], W_e2[i], b_e2[i][None])
        rn = _node_linear(r, W_n[i], b_n[i][None])

        pq = _msg_kernel()(rn, wij, i_src)
        m_ij = jnp.concatenate([pq[0, 0], pq[1, 0]], axis=1)
        m_ji = jnp.concatenate([pq[0, 1], pq[1, 1]], axis=1)
        agg_full = jax.ops.segment_sum(m_ij, a1p, num_segments=NPAD)
        agg_full = agg_full + jax.ops.segment_sum(m_ji, a0p, num_segments=NPAD)
        agg = jnp.stack([agg_full[:, :64], agg_full[:, 64:]])

        r = _update(agg, r, W_u1[i], b_u1[i][None], W_u2[i], b_u2[i][None])

    atom_e = _head(r, W_r1, b_r1[None], W_r2, b_r2[None])
    return atom_e[:N_MOL, 0]
